# baseline (speedup 1.0000x reference)
import jax
import jax.numpy as jnp
from jax import lax
from jax.experimental import pallas as pl
from jax.experimental.pallas import tpu as pltpu
from jax.experimental.pallas import tpu_sc as plsc

N = 100000
E = 3200000
NC = 2
NS = 16
NW = NC * NS
CHUNK = 128
CB = 6
BLK_E = CB * CHUNK
NBLK = 132
EW = NBLK * BLK_E
E_PAD = NW * EW
ACC_R = 100096
ZR = ACC_R // NS


def _sc_aggregate_body(t_hbm, e_hbm, z_hbm, out_hbm,
                       eA, rowsA, eB, rowsB, acc, gsA, gsB, ssA, ssB):
    cid = lax.axis_index("c")
    sid = lax.axis_index("s")
    wid = sid * NC + cid

    pltpu.sync_copy(z_hbm.at[pl.ds(sid * ZR, ZR)], acc.at[pl.ds(sid * ZR, ZR)])
    plsc.subcore_barrier()

    def load_fire(b, ebuf, rows, gs):
        pltpu.sync_copy(e_hbm.at[wid, b], ebuf)
        for j in range(CB):
            pltpu.async_copy(t_hbm.at[ebuf.at[j, 0]], rows.at[j], gs)

    def wait_gathers(ebuf, rows, gs):
        for j in range(CB):
            pltpu.make_async_copy(t_hbm.at[ebuf.at[j, 0]], rows.at[j], gs).wait()

    def scatter_drain(ebuf, rows, ss):
        for j in range(CB):
            pltpu.async_copy(rows.at[j], acc.at[ebuf.at[j, 1]], ss, add=True)
        for j in range(CB):
            pltpu.make_async_copy(rows.at[j], acc.at[ebuf.at[j, 1]], ss).wait()

    load_fire(0, eA, rowsA, gsA)

    def pair(p, carry):
        b0 = 2 * p
        load_fire(b0 + 1, eB, rowsB, gsB)
        wait_gathers(eA, rowsA, gsA)
        scatter_drain(eA, rowsA, ssA)
        load_fire(b0 + 2, eA, rowsA, gsA)
        wait_gathers(eB, rowsB, gsB)
        scatter_drain(eB, rowsB, ssB)
        return carry

    lax.fori_loop(0, NBLK // 2, pair, 0)
    wait_gathers(eA, rowsA, gsA)

    plsc.subcore_barrier()
    pltpu.sync_copy(acc.at[pl.ds(sid * ZR, ZR)],
                    out_hbm.at[cid].at[pl.ds(sid * ZR, ZR)])


def _sc_aggregate(table, edges5d, zeros_hbm):
    mesh = plsc.VectorSubcoreMesh(core_axis_name="c", subcore_axis_name="s")
    f = pl.kernel(
        _sc_aggregate_body,
        out_type=jax.ShapeDtypeStruct((NC, ACC_R, 16), jnp.float32),
        mesh=mesh,
        compiler_params=pltpu.CompilerParams(use_tc_tiling_on_sc=False),
        scratch_types=[
            pltpu.VMEM((CB, 2, CHUNK), jnp.int32),
            pltpu.VMEM((CB, CHUNK, 16), jnp.float32),
            pltpu.VMEM((CB, 2, CHUNK), jnp.int32),
            pltpu.VMEM((CB, CHUNK, 16), jnp.float32),
            pltpu.VMEM_SHARED((ACC_R, 16), jnp.float32),
            pltpu.SemaphoreType.DMA,
            pltpu.SemaphoreType.DMA,
            pltpu.SemaphoreType.DMA,
            pltpu.SemaphoreType.DMA,
        ],
    )
    return f(table, edges5d, zeros_hbm)


def _tc_layer1_body(p_ref, t_ref, wl_ref, wr_ref, b_ref, h_ref):
    s = p_ref[0] + p_ref[1]
    inv = 1.0 / jnp.maximum(s[:, 8:9], 1.0)
    mean = s * inv
    h = (jnp.dot(mean, wl_ref[...], preferred_element_type=jnp.float32)
         + jnp.dot(t_ref[...], wr_ref[...], preferred_element_type=jnp.float32)
         + b_ref[...])
    h_ref[...] = jnp.maximum(h, 0.0)


def _tc_layer2_body(p2_ref, p1_ref, h_ref, wl_ref, wr_ref, b_ref, o_ref):
    cnt = p1_ref[0][:, 8:9] + p1_ref[1][:, 8:9]
    inv = 1.0 / jnp.maximum(cnt, 1.0)
    s = p2_ref[0] + p2_ref[1]
    o_ref[...] = (jnp.dot(s * inv, wl_ref[...], preferred_element_type=jnp.float32)
                  + jnp.dot(h_ref[...], wr_ref[...], preferred_element_type=jnp.float32)
                  + b_ref[...])


_BN = 6256


def _tc_layer1(p1, table1, W1l_pad, W1r_pad, b1):
    return pl.pallas_call(
        _tc_layer1_body,
        grid=(ACC_R // _BN,),
        in_specs=[
            pl.BlockSpec((NC, _BN, 16), lambda i: (0, i, 0)),
            pl.BlockSpec((_BN, 16), lambda i: (i, 0)),
            pl.BlockSpec((16, 16), lambda i: (0, 0)),
            pl.BlockSpec((16, 16), lambda i: (0, 0)),
            pl.BlockSpec((1, 16), lambda i: (0, 0)),
        ],
        out_specs=pl.BlockSpec((_BN, 16), lambda i: (i, 0)),
        out_shape=jax.ShapeDtypeStruct((ACC_R, 16), jnp.float32),
    )(p1, table1, W1l_pad, W1r_pad, b1)


def _tc_layer2(p2, p1, h, W2_l, W2_r, b2):
    return pl.pallas_call(
        _tc_layer2_body,
        grid=(ACC_R // _BN,),
        in_specs=[
            pl.BlockSpec((NC, _BN, 16), lambda i: (0, i, 0)),
            pl.BlockSpec((NC, _BN, 16), lambda i: (0, i, 0)),
            pl.BlockSpec((_BN, 16), lambda i: (i, 0)),
            pl.BlockSpec((16, 16), lambda i: (0, 0)),
            pl.BlockSpec((16, 16), lambda i: (0, 0)),
            pl.BlockSpec((1, 16), lambda i: (0, 0)),
        ],
        out_specs=pl.BlockSpec((_BN, 16), lambda i: (i, 0)),
        out_shape=jax.ShapeDtypeStruct((ACC_R, 16), jnp.float32),
    )(p2, p1, h, W2_l, W2_r, b2)


def kernel(x, edge_index, W1_l, b1, W1_r, W2_l, b2, W2_r):
    src = edge_index[0]
    dst = edge_index[1]
    pad = E_PAD - E
    src_p = jnp.concatenate([src, jnp.zeros((pad,), jnp.int32)])
    dst_p = jnp.concatenate([dst, jnp.full((pad,), N, jnp.int32)])
    s4 = src_p.reshape(NW, NBLK, CB, CHUNK)
    d4 = dst_p.reshape(NW, NBLK, CB, CHUNK)
    e5 = jnp.stack([s4, d4], axis=3)
    e5 = jnp.pad(e5, ((0, 0), (0, 1), (0, 0), (0, 0), (0, 0)))
    zeros_hbm = jnp.zeros((ACC_R, 16), jnp.float32)

    table1 = jnp.concatenate(
        [x, jnp.ones((N, 1), jnp.float32), jnp.zeros((N, 7), jnp.float32)], axis=1)
    table1 = jnp.pad(table1, ((0, ACC_R - N), (0, 0)))
    W1l_pad = jnp.concatenate([W1_l, jnp.zeros((8, 16), jnp.float32)], axis=0)
    W1r_pad = jnp.concatenate([W1_r, jnp.zeros((8, 16), jnp.float32)], axis=0)

    p1 = _sc_aggregate(table1, e5, zeros_hbm)
    h = _tc_layer1(p1, table1, W1l_pad, W1r_pad, b1.reshape(1, 16))
    p2 = _sc_aggregate(h, e5, zeros_hbm)
    out = _tc_layer2(p2, p1, h, W2_l, W2_r, b2.reshape(1, 16))
    return out[:N]


# speedup vs baseline: 1.1711x; 1.1711x over previous
import jax
import jax.numpy as jnp
from jax import lax
from jax.experimental import pallas as pl
from jax.experimental.pallas import tpu as pltpu
from jax.experimental.pallas import tpu_sc as plsc

N = 100000
E = 3200000
NC = 2
NS = 16
NW = NC * NS
BLK_E = 512
NBLK = 196
EW = NBLK * BLK_E
E_PAD = NW * EW
ACC_R = 100096
ZR = ACC_R // NS


def _sc_aggregate_body(t_hbm, e_hbm, z_hbm, out_hbm,
                       sA, dA, rowsA, sB, dB, rowsB, acc, gsA, gsB, ssA, ssB):
    cid = lax.axis_index("c")
    sid = lax.axis_index("s")
    wid = sid * NC + cid

    pltpu.sync_copy(z_hbm.at[pl.ds(sid * ZR, ZR)], acc.at[pl.ds(sid * ZR, ZR)])
    plsc.subcore_barrier()

    def load_fire(b, sbuf, dbuf, rows, gs):
        pltpu.sync_copy(e_hbm.at[wid, b, 0], sbuf)
        pltpu.sync_copy(e_hbm.at[wid, b, 1], dbuf)
        pltpu.async_copy(t_hbm.at[sbuf], rows, gs)

    def wait_gathers(sbuf, rows, gs):
        pltpu.make_async_copy(t_hbm.at[sbuf], rows, gs).wait()

    def scatter_drain(dbuf, rows, ss):
        pltpu.async_copy(rows, acc.at[dbuf], ss, add=True)
        pltpu.make_async_copy(rows, acc.at[dbuf], ss).wait()

    load_fire(0, sA, dA, rowsA, gsA)

    def pair(p, carry):
        b0 = 2 * p
        load_fire(b0 + 1, sB, dB, rowsB, gsB)
        wait_gathers(sA, rowsA, gsA)
        scatter_drain(dA, rowsA, ssA)
        load_fire(b0 + 2, sA, dA, rowsA, gsA)
        wait_gathers(sB, rowsB, gsB)
        scatter_drain(dB, rowsB, ssB)
        return carry

    lax.fori_loop(0, NBLK // 2, pair, 0)
    wait_gathers(sA, rowsA, gsA)

    plsc.subcore_barrier()
    pltpu.sync_copy(acc.at[pl.ds(sid * ZR, ZR)],
                    out_hbm.at[cid].at[pl.ds(sid * ZR, ZR)])


def _sc_aggregate(table, edges5d, zeros_hbm):
    mesh = plsc.VectorSubcoreMesh(core_axis_name="c", subcore_axis_name="s")
    f = pl.kernel(
        _sc_aggregate_body,
        out_type=jax.ShapeDtypeStruct((NC, ACC_R, 16), jnp.float32),
        mesh=mesh,
        compiler_params=pltpu.CompilerParams(use_tc_tiling_on_sc=False),
        scratch_types=[
            pltpu.VMEM((BLK_E,), jnp.int32),
            pltpu.VMEM((BLK_E,), jnp.int32),
            pltpu.VMEM((BLK_E, 16), jnp.float32),
            pltpu.VMEM((BLK_E,), jnp.int32),
            pltpu.VMEM((BLK_E,), jnp.int32),
            pltpu.VMEM((BLK_E, 16), jnp.float32),
            pltpu.VMEM_SHARED((ACC_R, 16), jnp.float32),
            pltpu.SemaphoreType.DMA,
            pltpu.SemaphoreType.DMA,
            pltpu.SemaphoreType.DMA,
            pltpu.SemaphoreType.DMA,
        ],
    )
    return f(table, edges5d, zeros_hbm)


def _tc_layer1_body(p_ref, t_ref, wl_ref, wr_ref, b_ref, h_ref):
    s = p_ref[0] + p_ref[1]
    inv = 1.0 / jnp.maximum(s[:, 8:9], 1.0)
    mean = s * inv
    h = (jnp.dot(mean, wl_ref[...], preferred_element_type=jnp.float32)
         + jnp.dot(t_ref[...], wr_ref[...], preferred_element_type=jnp.float32)
         + b_ref[...])
    h_ref[...] = jnp.maximum(h, 0.0)


def _tc_layer2_body(p2_ref, p1_ref, h_ref, wl_ref, wr_ref, b_ref, o_ref):
    cnt = p1_ref[0][:, 8:9] + p1_ref[1][:, 8:9]
    inv = 1.0 / jnp.maximum(cnt, 1.0)
    s = p2_ref[0] + p2_ref[1]
    o_ref[...] = (jnp.dot(s * inv, wl_ref[...], preferred_element_type=jnp.float32)
                  + jnp.dot(h_ref[...], wr_ref[...], preferred_element_type=jnp.float32)
                  + b_ref[...])


_BN = 6256


def _tc_layer1(p1, table1, W1l_pad, W1r_pad, b1):
    return pl.pallas_call(
        _tc_layer1_body,
        grid=(ACC_R // _BN,),
        in_specs=[
            pl.BlockSpec((NC, _BN, 16), lambda i: (0, i, 0)),
            pl.BlockSpec((_BN, 16), lambda i: (i, 0)),
            pl.BlockSpec((16, 16), lambda i: (0, 0)),
            pl.BlockSpec((16, 16), lambda i: (0, 0)),
            pl.BlockSpec((1, 16), lambda i: (0, 0)),
        ],
        out_specs=pl.BlockSpec((_BN, 16), lambda i: (i, 0)),
        out_shape=jax.ShapeDtypeStruct((ACC_R, 16), jnp.float32),
    )(p1, table1, W1l_pad, W1r_pad, b1)


def _tc_layer2(p2, p1, h, W2_l, W2_r, b2):
    return pl.pallas_call(
        _tc_layer2_body,
        grid=(ACC_R // _BN,),
        in_specs=[
            pl.BlockSpec((NC, _BN, 16), lambda i: (0, i, 0)),
            pl.BlockSpec((NC, _BN, 16), lambda i: (0, i, 0)),
            pl.BlockSpec((_BN, 16), lambda i: (i, 0)),
            pl.BlockSpec((16, 16), lambda i: (0, 0)),
            pl.BlockSpec((16, 16), lambda i: (0, 0)),
            pl.BlockSpec((1, 16), lambda i: (0, 0)),
        ],
        out_specs=pl.BlockSpec((_BN, 16), lambda i: (i, 0)),
        out_shape=jax.ShapeDtypeStruct((ACC_R, 16), jnp.float32),
    )(p2, p1, h, W2_l, W2_r, b2)


def kernel(x, edge_index, W1_l, b1, W1_r, W2_l, b2, W2_r):
    src = edge_index[0]
    dst = edge_index[1]
    pad = E_PAD - E
    src_p = jnp.concatenate([src, jnp.zeros((pad,), jnp.int32)])
    dst_p = jnp.concatenate([dst, jnp.full((pad,), N, jnp.int32)])
    s4 = src_p.reshape(NW, NBLK, BLK_E)
    d4 = dst_p.reshape(NW, NBLK, BLK_E)
    e5 = jnp.stack([s4, d4], axis=2)
    e5 = jnp.pad(e5, ((0, 0), (0, 1), (0, 0), (0, 0)))
    zeros_hbm = jnp.zeros((ACC_R, 16), jnp.float32)

    table1 = jnp.concatenate(
        [x, jnp.ones((N, 1), jnp.float32), jnp.zeros((N, 7), jnp.float32)], axis=1)
    table1 = jnp.pad(table1, ((0, ACC_R - N), (0, 0)))
    W1l_pad = jnp.concatenate([W1_l, jnp.zeros((8, 16), jnp.float32)], axis=0)
    W1r_pad = jnp.concatenate([W1_r, jnp.zeros((8, 16), jnp.float32)], axis=0)

    p1 = _sc_aggregate(table1, e5, zeros_hbm)
    h = _tc_layer1(p1, table1, W1l_pad, W1r_pad, b1.reshape(1, 16))
    p2 = _sc_aggregate(h, e5, zeros_hbm)
    out = _tc_layer2(p2, p1, h, W2_l, W2_r, b2.reshape(1, 16))
    return out[:N]
